# Initial kernel scaffold; baseline (speedup 1.0000x reference)
import jax
import jax.numpy as jnp
from jax.experimental import pallas as pl


def kernel(mu, log_sigma, edge_index_2xE, batch_size):
    raise NotImplementedError("write your pallas kernel here")


# speedup vs baseline: 1.9120x; 1.9120x over previous
import functools
import math

import jax
import jax.numpy as jnp
from jax import lax
from jax.experimental import pallas as pl
from jax.experimental.pallas import tpu as pltpu
from jax.experimental.pallas import tpu_sc as plsc

_LANES = 16
_CH = 128


@functools.lru_cache(maxsize=None)
def _make_sc_d2(n_nodes, d_dim, n_chunks, nc, ns):
    nw = nc * ns
    e_pad = nw * n_chunks * _CH
    per_w = n_chunks * _CH
    mesh = plsc.VectorSubcoreMesh(core_axis_name="c", subcore_axis_name="s")

    @functools.partial(
        pl.kernel,
        mesh=mesh,
        out_type=jax.ShapeDtypeStruct((e_pad,), jnp.float32),
        scratch_types=[
            pltpu.VMEM((n_chunks, _CH), jnp.int32),
            pltpu.VMEM((n_chunks, _CH), jnp.int32),
            pltpu.VMEM((_CH, d_dim), jnp.float32),
            pltpu.VMEM((_CH, d_dim), jnp.float32),
            pltpu.VMEM((per_w,), jnp.float32),
            pltpu.SemaphoreType.DMA,
            pltpu.SemaphoreType.DMA,
        ],
        compiler_params=pltpu.CompilerParams(
            needs_layout_passes=False, use_tc_tiling_on_sc=False),
    )
    def sc_d2(mu_hbm, idxi_hbm, idxj_hbm, out_hbm,
              idxi_v, idxj_v, rowsi_v, rowsj_v, d2_v, sem_i, sem_j):
        cid = lax.axis_index("c")
        sid = lax.axis_index("s")
        wid = sid * nc + cid
        pltpu.sync_copy(idxi_hbm.at[wid], idxi_v)
        pltpu.sync_copy(idxj_hbm.at[wid], idxj_v)
        lanes = jnp.arange(_LANES, dtype=jnp.int32)

        def chunk_body(c, carry):
            cp_i = pltpu.make_async_copy(
                mu_hbm.at[idxi_v.at[c]], rowsi_v, sem_i)
            cp_j = pltpu.make_async_copy(
                mu_hbm.at[idxj_v.at[c]], rowsj_v, sem_j)
            cp_i.start()
            cp_j.start()
            cp_i.wait()
            cp_j.wait()
            for g in range(_CH // _LANES):
                ids = lanes + (g * _LANES)
                acc = jnp.zeros((_LANES,), jnp.float32)
                for k in range(d_dim):
                    kk = jnp.full((_LANES,), k, jnp.int32)
                    vi = plsc.load_gather(rowsi_v, [ids, kk])
                    vj = plsc.load_gather(rowsj_v, [ids, kk])
                    dv = vi - vj
                    acc = acc + dv * dv
                d2_v[pl.ds(c * _CH + g * _LANES, _LANES)] = acc
            return carry

        lax.fori_loop(0, n_chunks, chunk_body, 0)
        pltpu.sync_copy(d2_v, out_hbm.at[pl.ds(wid * per_w, per_w)])

    return sc_d2


def _tc_reduce_body(pad_fix, d2_ref, mu_ref, ls_ref, out_ref):
    g = pl.program_id(0)
    x = d2_ref[...]
    ll = jnp.sum(jax.nn.log_sigmoid(-x))
    sigma = jax.nn.softplus(ls_ref[...]) + 1e-6
    mu_v = mu_ref[...]
    kl = 0.5 * jnp.sum(sigma * sigma + mu_v * mu_v - 1.0 - 2.0 * jnp.log(sigma))
    part = ll - kl

    @pl.when(g == 0)
    def _init():
        out_ref[0, 0] = part + pad_fix

    @pl.when(g != 0)
    def _acc():
        out_ref[0, 0] += part


def kernel(mu, log_sigma, edge_index_2xE, batch_size):
    n_nodes, d_dim = mu.shape
    e = edge_index_2xE.shape[1]
    info = plsc.get_sparse_core_info()
    nc, ns = info.num_cores, info.num_subcores
    nw = nc * ns

    per_round = nw * _CH
    n_chunks = -(-e // per_round)
    e_pad = n_chunks * per_round
    pad = e_pad - e

    idx = edge_index_2xE.astype(jnp.int32)
    zpad = jnp.zeros((pad,), jnp.int32)
    idx_i = jnp.concatenate([idx[0], zpad]).reshape(nw, n_chunks, _CH)
    idx_j = jnp.concatenate([idx[1], zpad]).reshape(nw, n_chunks, _CH)

    sc_d2 = _make_sc_d2(n_nodes, d_dim, n_chunks, nc, ns)
    d2 = sc_d2(mu, idx_i, idx_j)

    g_grid = 8
    wide = 128 * g_grid
    a_rows = e_pad // wide
    b_rows = (n_nodes * d_dim) // wide

    d2_2d = d2.reshape(a_rows, wide)
    mu_2d = mu.reshape(b_rows, wide)
    ls_2d = log_sigma.reshape(b_rows, wide)
    pad_fix = float(pad) * math.log(2.0)

    out = pl.pallas_call(
        functools.partial(_tc_reduce_body, pad_fix),
        grid=(g_grid,),
        in_specs=[
            pl.BlockSpec((a_rows, 128), lambda g: (0, g)),
            pl.BlockSpec((b_rows, 128), lambda g: (0, g)),
            pl.BlockSpec((b_rows, 128), lambda g: (0, g)),
        ],
        out_specs=pl.BlockSpec(memory_space=pltpu.SMEM),
        out_shape=jax.ShapeDtypeStruct((1, 1), jnp.float32),
    )(d2_2d, mu_2d, ls_2d)
    return out[0, 0]
